# baseline (speedup 1.0000x reference)
import functools

import jax
import jax.numpy as jnp
from jax import lax
from jax.experimental import pallas as pl
from jax.experimental.pallas import tpu as pltpu
from jax.experimental.pallas import tpu_sc as plsc

_B = 4096
_H = 50
_HP = 64
_D = 128
_LANES = 16
_NCHUNK = _D // _LANES
_NBUF = 2


def _make_kernel():
    info = plsc.get_sparse_core_info()
    nc, ns = info.num_cores, info.num_subcores
    nw = nc * ns
    bpw = _B // nw

    mesh = plsc.VectorSubcoreMesh(core_axis_name="c", subcore_axis_name="s")

    @functools.partial(
        pl.kernel,
        mesh=mesh,
        compiler_params=pltpu.CompilerParams(
            needs_layout_passes=False, use_tc_tiling_on_sc=False),
        out_type=jax.ShapeDtypeStruct((_B, _D), jnp.float32),
        scratch_types=[
            pltpu.VMEM((bpw, _H), jnp.int32),
            pltpu.VMEM((bpw, _HP), jnp.float32),
            pltpu.VMEM((bpw, _D), jnp.float32),
        ] + [pltpu.VMEM((_H, _D // 2), jnp.int32) for _ in range(_NBUF)]
          + [pltpu.SemaphoreType.DMA for _ in range(_NBUF)],
    )
    def emb_kernel(x_hbm, w_hbm, tbl_hbm, out_hbm, xv, wv, outv, *ring):
        bufs, sems = ring[:_NBUF], ring[_NBUF:]
        wid = lax.axis_index("s") * nc + lax.axis_index("c")
        base = wid * bpw
        pltpu.sync_copy(x_hbm.at[pl.ds(base, bpw)], xv)
        pltpu.sync_copy(w_hbm.at[pl.ds(base, bpw)], wv)

        mask = jnp.full((_LANES,), 0xFFFF0000, dtype=jnp.uint32)

        def compute(b, rows):
            wrow = [wv[b, pl.ds(g * _LANES, _LANES)] for g in range(_H // _LANES + 1)]
            acc_ev = [jnp.zeros((_LANES,), jnp.float32) for _ in range(_NCHUNK // 2)]
            acc_od = [jnp.zeros((_LANES,), jnp.float32) for _ in range(_NCHUNK // 2)]
            for l in range(_H):
                wb = jnp.broadcast_to(wrow[l // _LANES][l % _LANES], (_LANES,))
                for c in range(_NCHUNK // 2):
                    u = plsc.bitcast(rows[l, pl.ds(c * _LANES, _LANES)], jnp.uint32)
                    f_ev = plsc.bitcast(u << 16, jnp.float32)
                    f_od = plsc.bitcast(u & mask, jnp.float32)
                    acc_ev[c] = acc_ev[c] + wb * f_ev
                    acc_od[c] = acc_od[c] + wb * f_od
            for c in range(_NCHUNK // 2):
                outv[b, pl.ds(c * 32, _LANES)] = acc_ev[c]
                outv[b, pl.ds(c * 32 + _LANES, _LANES)] = acc_od[c]

        for k in range(_NBUF):
            pltpu.async_copy(tbl_hbm.at[xv.at[k]], bufs[k], sems[k])

        def body(g, _):
            b0 = _NBUF * g
            for k in range(_NBUF):
                pltpu.make_async_copy(tbl_hbm.at[xv.at[0]], bufs[k], sems[k]).wait()
                compute(b0 + k, bufs[k])
                pltpu.async_copy(tbl_hbm.at[xv.at[b0 + k + _NBUF]], bufs[k], sems[k])
            return 0

        lax.fori_loop(0, bpw // _NBUF - 1, body, 0)
        for k in range(_NBUF):
            pltpu.make_async_copy(tbl_hbm.at[xv.at[0]], bufs[k], sems[k]).wait()
            compute(bpw - _NBUF + k, bufs[k])
        pltpu.sync_copy(outv, out_hbm.at[pl.ds(base, bpw)])

    return emb_kernel


def kernel(x, w, weight):
    wp = jnp.pad(w, ((0, 0), (0, _HP - _H)))
    tbl16 = weight.astype(jnp.bfloat16)
    tblp = jax.lax.bitcast_convert_type(
        tbl16.reshape(tbl16.shape[0], _D // 2, 2), jnp.int32)
    perm = _make_kernel()(x.astype(jnp.int32), wp, tblp)
    return (
        perm.reshape(_B, _NCHUNK // 2, 2, _LANES)
        .transpose(0, 1, 3, 2)
        .reshape(_B, _D)
    )


# speedup vs baseline: 4.5697x; 4.5697x over previous
import functools

import jax
import jax.numpy as jnp
from jax import lax
from jax.experimental import pallas as pl
from jax.experimental.pallas import tpu as pltpu
from jax.experimental.pallas import tpu_sc as plsc

_B = 4096
_H = 50
_HP = 64
_D = 128
_LANES = 16
_NCHUNK = _D // _LANES
_NBUF = 3


def _make_kernel():
    info = plsc.get_sparse_core_info()
    nc, ns = info.num_cores, info.num_subcores
    nw = nc * ns
    bpw = _B // nw

    mesh = plsc.VectorSubcoreMesh(core_axis_name="c", subcore_axis_name="s")

    @functools.partial(
        pl.kernel,
        mesh=mesh,
        out_type=jax.ShapeDtypeStruct((_B, _D), jnp.float32),
        scratch_types=[
            pltpu.VMEM((bpw, _H), jnp.int32),
            pltpu.VMEM((bpw, _HP), jnp.float32),
            pltpu.VMEM((bpw, _D), jnp.float32),
        ] + [pltpu.VMEM((_H, _D), jnp.float32) for _ in range(_NBUF)]
          + [pltpu.SemaphoreType.DMA for _ in range(_NBUF)],
    )
    def emb_kernel(x_hbm, w_hbm, tbl_hbm, out_hbm, xv, wv, outv, *ring):
        bufs, sems = ring[:_NBUF], ring[_NBUF:]
        wid = lax.axis_index("s") * nc + lax.axis_index("c")
        base = wid * bpw
        pltpu.sync_copy(x_hbm.at[pl.ds(base, bpw)], xv)
        pltpu.sync_copy(w_hbm.at[pl.ds(base, bpw)], wv)

        def compute(b, rows):
            wrow = [wv[b, pl.ds(g * _LANES, _LANES)] for g in range(_H // _LANES + 1)]
            accs = [jnp.zeros((_LANES,), jnp.float32) for _ in range(_NCHUNK)]
            for l in range(_H):
                wb = jnp.broadcast_to(wrow[l // _LANES][l % _LANES], (_LANES,))
                for c in range(_NCHUNK):
                    accs[c] = accs[c] + wb * rows[l, pl.ds(c * _LANES, _LANES)]
            for c in range(_NCHUNK):
                outv[b, pl.ds(c * _LANES, _LANES)] = accs[c]

        pltpu.async_copy(tbl_hbm.at[xv.at[0]], bufs[0], sems[0])
        pltpu.async_copy(tbl_hbm.at[xv.at[1]], bufs[1], sems[1])

        def body(g, _):
            b0 = 3 * g
            for k in range(3):
                pltpu.make_async_copy(tbl_hbm.at[xv.at[0]], bufs[k], sems[k]).wait()
                nxt = (k + 2) % 3
                pltpu.async_copy(
                    tbl_hbm.at[xv.at[b0 + k + 2]], bufs[nxt], sems[nxt])
                compute(b0 + k, bufs[k])
            return 0

        lax.fori_loop(0, (bpw - 2) // 3, body, 0)
        pltpu.make_async_copy(tbl_hbm.at[xv.at[0]], bufs[0], sems[0]).wait()
        compute(bpw - 2, bufs[0])
        pltpu.make_async_copy(tbl_hbm.at[xv.at[0]], bufs[1], sems[1]).wait()
        compute(bpw - 1, bufs[1])
        pltpu.sync_copy(outv, out_hbm.at[pl.ds(base, bpw)])

    return emb_kernel


def kernel(x, w, weight):
    wp = jnp.pad(w, ((0, 0), (0, _HP - _H)))
    return _make_kernel()(x.astype(jnp.int32), wp, weight)


# speedup vs baseline: 5.4344x; 1.1892x over previous
import functools

import jax
import jax.numpy as jnp
from jax import lax
from jax.experimental import pallas as pl
from jax.experimental.pallas import tpu as pltpu
from jax.experimental.pallas import tpu_sc as plsc

_B = 4096
_H = 50
_HP = 64
_D = 128
_LANES = 16
_NCHUNK = _D // _LANES
_NBUF = 2
_UNROLL = 10


def _make_kernel():
    info = plsc.get_sparse_core_info()
    nc, ns = info.num_cores, info.num_subcores
    nw = nc * ns
    bpw = _B // nw

    mesh = plsc.VectorSubcoreMesh(core_axis_name="c", subcore_axis_name="s")

    @functools.partial(
        pl.kernel,
        mesh=mesh,
        compiler_params=pltpu.CompilerParams(needs_layout_passes=False),
        out_type=jax.ShapeDtypeStruct((_B, _D), jnp.float32),
        scratch_types=[
            pltpu.VMEM((bpw, _H), jnp.int32),
            pltpu.VMEM((bpw, _HP), jnp.float32),
            pltpu.VMEM((bpw, _D), jnp.float32),
        ] + [pltpu.VMEM((_H, _D), jnp.float32) for _ in range(_NBUF)]
          + [pltpu.SemaphoreType.DMA for _ in range(_NBUF)],
    )
    def emb_kernel(x_hbm, w_hbm, tbl_hbm, out_hbm, xv, wv, outv, *ring):
        bufs, sems = ring[:_NBUF], ring[_NBUF:]
        wid = lax.axis_index("s") * nc + lax.axis_index("c")
        base = wid * bpw
        pltpu.sync_copy(x_hbm.at[pl.ds(base, bpw)], xv)
        pltpu.sync_copy(w_hbm.at[pl.ds(base, bpw)], wv)

        def compute(b, rows):
            b16 = jnp.broadcast_to(b, (_LANES,))

            def tok(t, accs):
                res = list(accs)
                for dl in range(_UNROLL):
                    l = t * _UNROLL + dl
                    wb = plsc.load_gather(
                        wv, [b16, jnp.broadcast_to(l, (_LANES,))])
                    for c in range(_NCHUNK):
                        res[c] = res[c] + wb * rows[l, pl.ds(c * _LANES, _LANES)]
                return tuple(res)

            accs = lax.fori_loop(
                0, _H // _UNROLL, tok,
                tuple(jnp.zeros((_LANES,), jnp.float32) for _ in range(_NCHUNK)),
            )
            for c in range(_NCHUNK):
                outv[b, pl.ds(c * _LANES, _LANES)] = accs[c]

        for k in range(_NBUF):
            pltpu.async_copy(tbl_hbm.at[xv.at[k]], bufs[k], sems[k])

        def body(g, _):
            b0 = _NBUF * g
            for k in range(_NBUF):
                pltpu.make_async_copy(tbl_hbm.at[xv.at[0]], bufs[k], sems[k]).wait()
                compute(b0 + k, bufs[k])
                pltpu.async_copy(tbl_hbm.at[xv.at[b0 + k + _NBUF]], bufs[k], sems[k])
            return 0

        lax.fori_loop(0, bpw // _NBUF - 1, body, 0)
        for k in range(_NBUF):
            pltpu.make_async_copy(tbl_hbm.at[xv.at[0]], bufs[k], sems[k]).wait()
            compute(bpw - _NBUF + k, bufs[k])
        pltpu.sync_copy(outv, out_hbm.at[pl.ds(base, bpw)])

    return emb_kernel


def kernel(x, w, weight):
    wp = jnp.pad(w, ((0, 0), (0, _HP - _H)))
    return _make_kernel()(x.astype(jnp.int32), wp, weight)


# speedup vs baseline: 6.3778x; 1.1736x over previous
import functools

import jax
import jax.numpy as jnp
from jax import lax
from jax.experimental import pallas as pl
from jax.experimental.pallas import tpu as pltpu
from jax.experimental.pallas import tpu_sc as plsc

_B = 4096
_H = 50
_HP = 64
_D = 128
_LANES = 16
_NCHUNK = _D // _LANES
_NBUF = 3
_UNROLL = 10


def _make_kernel():
    info = plsc.get_sparse_core_info()
    nc, ns = info.num_cores, info.num_subcores
    nw = nc * ns
    bpw = _B // nw

    mesh = plsc.VectorSubcoreMesh(core_axis_name="c", subcore_axis_name="s")

    @functools.partial(
        pl.kernel,
        mesh=mesh,
        compiler_params=pltpu.CompilerParams(needs_layout_passes=False),
        out_type=jax.ShapeDtypeStruct((_B, _D), jnp.float32),
        scratch_types=[
            pltpu.VMEM((bpw, _H), jnp.int32),
            pltpu.VMEM((bpw, _HP), jnp.float32),
            pltpu.VMEM((bpw, _D), jnp.float32),
        ] + [pltpu.VMEM((_H, _D), jnp.float32) for _ in range(_NBUF)]
          + [pltpu.SemaphoreType.DMA for _ in range(_NBUF)],
    )
    def emb_kernel(x_hbm, w_hbm, tbl_hbm, out_hbm, xv, wv, outv, *ring):
        bufs, sems = ring[:_NBUF], ring[_NBUF:]
        wid = lax.axis_index("s") * nc + lax.axis_index("c")
        base = wid * bpw
        pltpu.sync_copy(x_hbm.at[pl.ds(base, bpw)], xv)
        pltpu.sync_copy(w_hbm.at[pl.ds(base, bpw)], wv)

        def compute(b, rows):
            b16 = jnp.broadcast_to(b, (_LANES,))

            def tok(t, accs):
                res = list(accs)
                for dl in range(_UNROLL):
                    l = t * _UNROLL + dl
                    wb = plsc.load_gather(
                        wv, [b16, jnp.broadcast_to(l, (_LANES,))])
                    for c in range(_NCHUNK):
                        res[c] = res[c] + wb * rows[l, pl.ds(c * _LANES, _LANES)]
                return tuple(res)

            accs = lax.fori_loop(
                0, _H // _UNROLL, tok,
                tuple(jnp.zeros((_LANES,), jnp.float32) for _ in range(_NCHUNK)),
            )
            for c in range(_NCHUNK):
                outv[b, pl.ds(c * _LANES, _LANES)] = accs[c]

        pltpu.async_copy(tbl_hbm.at[xv.at[0]], bufs[0], sems[0])
        pltpu.async_copy(tbl_hbm.at[xv.at[1]], bufs[1], sems[1])

        def body(g, _):
            b0 = 3 * g
            for k in range(3):
                pltpu.make_async_copy(tbl_hbm.at[xv.at[0]], bufs[k], sems[k]).wait()
                nxt = (k + 2) % 3
                pltpu.async_copy(
                    tbl_hbm.at[xv.at[b0 + k + 2]], bufs[nxt], sems[nxt])
                compute(b0 + k, bufs[k])
            return 0

        lax.fori_loop(0, (bpw - 2) // 3, body, 0)
        pltpu.make_async_copy(tbl_hbm.at[xv.at[0]], bufs[0], sems[0]).wait()
        compute(bpw - 2, bufs[0])
        pltpu.make_async_copy(tbl_hbm.at[xv.at[0]], bufs[1], sems[1]).wait()
        compute(bpw - 1, bufs[1])
        pltpu.sync_copy(outv, out_hbm.at[pl.ds(base, bpw)])

    return emb_kernel


def kernel(x, w, weight):
    wp = jnp.pad(w, ((0, 0), (0, _HP - _H)))
    return _make_kernel()(x.astype(jnp.int32), wp, weight)


# speedup vs baseline: 7.6966x; 1.2068x over previous
import functools

import jax
import jax.numpy as jnp
from jax import lax
from jax.experimental import pallas as pl
from jax.experimental.pallas import tpu as pltpu
from jax.experimental.pallas import tpu_sc as plsc

_B = 4096
_H = 50
_HP = 64
_D = 128
_LANES = 16
_NCHUNK = _D // _LANES
_NBUF = 4
_UNROLL = 10


def _make_kernel():
    info = plsc.get_sparse_core_info()
    nc, ns = info.num_cores, info.num_subcores
    nw = nc * ns
    bpw = _B // nw

    mesh = plsc.VectorSubcoreMesh(core_axis_name="c", subcore_axis_name="s")

    @functools.partial(
        pl.kernel,
        mesh=mesh,
        compiler_params=pltpu.CompilerParams(needs_layout_passes=False),
        out_type=jax.ShapeDtypeStruct((_B, _D), jnp.float32),
        scratch_types=[
            pltpu.VMEM((bpw, _H), jnp.int32),
            pltpu.VMEM((bpw, _HP), jnp.float32),
            pltpu.VMEM((bpw, _D), jnp.float32),
        ] + [pltpu.VMEM((_H, _D), jnp.float32) for _ in range(_NBUF)]
          + [pltpu.SemaphoreType.DMA for _ in range(_NBUF)],
    )
    def emb_kernel(x_hbm, w_hbm, tbl_hbm, out_hbm, xv, wv, outv, *ring):
        bufs, sems = ring[:_NBUF], ring[_NBUF:]
        wid = lax.axis_index("s") * nc + lax.axis_index("c")
        base = wid * bpw
        pltpu.sync_copy(x_hbm.at[pl.ds(base, bpw)], xv)
        pltpu.sync_copy(w_hbm.at[pl.ds(base, bpw)], wv)

        def compute(b, rows):
            b16 = jnp.broadcast_to(b, (_LANES,))

            def tok(t, accs):
                res = list(accs)
                for dl in range(_UNROLL):
                    l = t * _UNROLL + dl
                    wb = plsc.load_gather(
                        wv, [b16, jnp.broadcast_to(l, (_LANES,))])
                    for c in range(_NCHUNK):
                        res[c] = res[c] + wb * rows[l, pl.ds(c * _LANES, _LANES)]
                return tuple(res)

            accs = lax.fori_loop(
                0, _H // _UNROLL, tok,
                tuple(jnp.zeros((_LANES,), jnp.float32) for _ in range(_NCHUNK)),
            )
            for c in range(_NCHUNK):
                outv[b, pl.ds(c * _LANES, _LANES)] = accs[c]

        def wait(k):
            pltpu.make_async_copy(tbl_hbm.at[xv.at[0]], bufs[k], sems[k]).wait()

        for k in range(_NBUF - 1):
            pltpu.async_copy(tbl_hbm.at[xv.at[k]], bufs[k], sems[k])

        n_loop = (bpw - (_NBUF - 1)) // _NBUF

        def body(g, _):
            b0 = _NBUF * g
            for k in range(_NBUF):
                wait(k)
                nxt = (k + _NBUF - 1) % _NBUF
                pltpu.async_copy(
                    tbl_hbm.at[xv.at[b0 + k + _NBUF - 1]], bufs[nxt], sems[nxt])
                compute(b0 + k, bufs[k])
            return 0

        lax.fori_loop(0, n_loop, body, 0)
        issued_max = _NBUF * n_loop + _NBUF - 2
        for r in range(_NBUF * n_loop, bpw):
            ri = r + _NBUF - 1
            if ri < bpw and ri > issued_max:
                pltpu.async_copy(
                    tbl_hbm.at[xv.at[ri]], bufs[ri % _NBUF], sems[ri % _NBUF])
            wait(r % _NBUF)
            compute(r, bufs[r % _NBUF])
        pltpu.sync_copy(outv, out_hbm.at[pl.ds(base, bpw)])

    return emb_kernel


def kernel(x, w, weight):
    wp = jnp.pad(w, ((0, 0), (0, _HP - _H)))
    return _make_kernel()(x.astype(jnp.int32), wp, weight)


# speedup vs baseline: 8.2120x; 1.0670x over previous
import functools

import jax
import jax.numpy as jnp
from jax import lax
from jax.experimental import pallas as pl
from jax.experimental.pallas import tpu as pltpu
from jax.experimental.pallas import tpu_sc as plsc

_B = 4096
_H = 50
_HP = 64
_D = 128
_LANES = 16
_NCHUNK = _D // _LANES
_NBUF = 5
_UNROLL = 10


def _make_kernel():
    info = plsc.get_sparse_core_info()
    nc, ns = info.num_cores, info.num_subcores
    nw = nc * ns
    bpw = _B // nw

    mesh = plsc.VectorSubcoreMesh(core_axis_name="c", subcore_axis_name="s")

    @functools.partial(
        pl.kernel,
        mesh=mesh,
        compiler_params=pltpu.CompilerParams(needs_layout_passes=False),
        out_type=jax.ShapeDtypeStruct((_B, _D), jnp.float32),
        scratch_types=[
            pltpu.VMEM((bpw, _H), jnp.int32),
            pltpu.VMEM((bpw, _HP), jnp.float32),
            pltpu.VMEM((bpw, _D), jnp.float32),
        ] + [pltpu.VMEM((_H, _D), jnp.float32) for _ in range(_NBUF)]
          + [pltpu.SemaphoreType.DMA for _ in range(_NBUF)],
    )
    def emb_kernel(x_hbm, w_hbm, tbl_hbm, out_hbm, xv, wv, outv, *ring):
        bufs, sems = ring[:_NBUF], ring[_NBUF:]
        wid = lax.axis_index("s") * nc + lax.axis_index("c")
        base = wid * bpw
        pltpu.sync_copy(x_hbm.at[pl.ds(base, bpw)], xv)
        pltpu.sync_copy(w_hbm.at[pl.ds(base, bpw)], wv)

        def compute(b, rows):
            b16 = jnp.broadcast_to(b, (_LANES,))

            def tok(t, accs):
                res = list(accs)
                for dl in range(_UNROLL):
                    l = t * _UNROLL + dl
                    wb = plsc.load_gather(
                        wv, [b16, jnp.broadcast_to(l, (_LANES,))])
                    for c in range(_NCHUNK):
                        res[c] = res[c] + wb * rows[l, pl.ds(c * _LANES, _LANES)]
                return tuple(res)

            accs = lax.fori_loop(
                0, _H // _UNROLL, tok,
                tuple(jnp.zeros((_LANES,), jnp.float32) for _ in range(_NCHUNK)),
            )
            for c in range(_NCHUNK):
                outv[b, pl.ds(c * _LANES, _LANES)] = accs[c]

        def wait(k):
            pltpu.make_async_copy(tbl_hbm.at[xv.at[0]], bufs[k], sems[k]).wait()

        for k in range(_NBUF - 1):
            pltpu.async_copy(tbl_hbm.at[xv.at[k]], bufs[k], sems[k])

        n_loop = (bpw - (_NBUF - 1)) // _NBUF

        def body(g, _):
            b0 = _NBUF * g
            for k in range(_NBUF):
                wait(k)
                nxt = (k + _NBUF - 1) % _NBUF
                pltpu.async_copy(
                    tbl_hbm.at[xv.at[b0 + k + _NBUF - 1]], bufs[nxt], sems[nxt])
                compute(b0 + k, bufs[k])
            return 0

        lax.fori_loop(0, n_loop, body, 0)
        issued_max = _NBUF * n_loop + _NBUF - 2
        for r in range(_NBUF * n_loop, bpw):
            ri = r + _NBUF - 1
            if ri < bpw and ri > issued_max:
                pltpu.async_copy(
                    tbl_hbm.at[xv.at[ri]], bufs[ri % _NBUF], sems[ri % _NBUF])
            wait(r % _NBUF)
            compute(r, bufs[r % _NBUF])
        pltpu.sync_copy(outv, out_hbm.at[pl.ds(base, bpw)])

    return emb_kernel


def kernel(x, w, weight):
    wp = jnp.pad(w, ((0, 0), (0, _HP - _H)))
    return _make_kernel()(x.astype(jnp.int32), wp, weight)
